# baseline (speedup 1.0000x reference)
import jax
import jax.numpy as jnp
from jax import lax
from jax.experimental import pallas as pl
from jax.experimental.pallas import tpu as pltpu
from jax.experimental.pallas import tpu_sc as plsc

_F = 26
_K = 16
_B = 4096
_H = 1000000
_NC = 2
_NS = 16
_NW = _NC * _NS
_BPW = _B // _NW
_G = _BPW // _K

_RUNROWS = 2048
_NFULL = _H // _RUNROWS
_TAILROWS = 512
_LAST = 64
_TAILWID = _NFULL % _NW


def _transpose_body(src_ref, tail_ref, out_ref, in0a, in0b, in1a, in1b,
                    ob0, ob1, is0, is1, os0, os1):
    wid = lax.axis_index("s") * _NC + lax.axis_index("c")
    lane = lax.iota(jnp.int32, _K)
    lanex16 = lane * _K

    def transpose_panel(nrows, ina, inb, ob):
        idxvecs = [lanex16 + k for k in range(_K)]

        def blk(c0i, carry):
            c0 = c0i * _K
            obs = ob.at[pl.ds(c0i * 256, 256)]
            for k in range(_K):
                src = ina if k < 8 else inb
                v = src[k % 8, pl.ds(c0, _K)]
                plsc.store_scatter(obs, [idxvecs[k]], v)
            return carry
        lax.fori_loop(0, nrows // _K, blk, 0, unroll=4)

    inbufs = ((in0a, in0b), (in1a, in1b))
    obufs = (ob0, ob1)
    isems = (is0, is1)
    osems = (os0, os1)
    nj = (_NFULL + _NW - 1) // _NW

    def start_in(run, b):
        @pl.when(run < _NFULL)
        def _():
            sl = pl.ds(run * _RUNROWS, _RUNROWS)
            pltpu.async_copy(src_ref.at[pl.ds(0, 8), sl], inbufs[b][0], isems[b])
            pltpu.async_copy(src_ref.at[pl.ds(8, 8), sl], inbufs[b][1], isems[b])

    def do_phase(j, b):
        run = wid + _NW * j
        nxt = run + _NW

        @pl.when(nxt < _NFULL)
        def _():
            sl = pl.ds(nxt * _RUNROWS, _RUNROWS)
            pltpu.async_copy(src_ref.at[pl.ds(0, 8), sl], inbufs[1 - b][0], isems[1 - b])
            pltpu.async_copy(src_ref.at[pl.ds(8, 8), sl], inbufs[1 - b][1], isems[1 - b])

        if j >= 2:
            @pl.when(run - 2 * _NW < _NFULL)
            def _():
                pltpu.make_async_copy(
                    obufs[b],
                    out_ref.at[pl.ds((run - 2 * _NW) * _RUNROWS * _K,
                                     _RUNROWS * _K)], osems[b]).wait()

        @pl.when(run < _NFULL)
        def _():
            sl = pl.ds(run * _RUNROWS, _RUNROWS)
            pltpu.make_async_copy(
                src_ref.at[pl.ds(0, 8), sl], inbufs[b][0], isems[b]).wait()
            pltpu.make_async_copy(
                src_ref.at[pl.ds(8, 8), sl], inbufs[b][1], isems[b]).wait()
            transpose_panel(_RUNROWS, inbufs[b][0], inbufs[b][1], obufs[b])
            pltpu.async_copy(
                obufs[b], out_ref.at[pl.ds(run * _RUNROWS * _K, _RUNROWS * _K)],
                osems[b])

    start_in(wid, 0)
    for j in range(nj):
        do_phase(j, j % 2)
    for j in (nj - 2, nj - 1):
        run = wid + _NW * j

        @pl.when(run < _NFULL)
        def _():
            pltpu.make_async_copy(
                obufs[j % 2],
                out_ref.at[pl.ds(run * _RUNROWS * _K, _RUNROWS * _K)],
                osems[j % 2]).wait()

    @pl.when(wid == _TAILWID)
    def _tail():
        r0 = _NFULL * _RUNROWS
        tsl = pl.ds(r0, _TAILROWS)
        pltpu.async_copy(
            src_ref.at[pl.ds(0, 8), tsl],
            in0a.at[:, pl.ds(0, _TAILROWS)], is0).wait()
        pltpu.async_copy(
            src_ref.at[pl.ds(8, 8), tsl],
            in0b.at[:, pl.ds(0, _TAILROWS)], is0).wait()
        transpose_panel(_TAILROWS, in0a, in0b, ob0)
        pltpu.async_copy(
            ob0.at[pl.ds(0, _TAILROWS * _K)],
            out_ref.at[pl.ds(r0 * _K, _TAILROWS * _K)], os0).wait()
        pltpu.async_copy(tail_ref, ob1.at[pl.ds(0, _LAST * _K)], is1).wait()
        pltpu.async_copy(
            ob1.at[pl.ds(0, _LAST * _K)],
            out_ref.at[pl.ds((_H - _LAST) * _K, _LAST * _K)], os1).wait()


def _fm_body(x_ref, emb_ref, w1_ref, w0_ref, fm_ref, prob_ref,
             idx_v, idxq_v, rows_v, w1r_v, w0_v, rbuf_v, out_v, sem, wsem):
    wid = lax.axis_index("s") * _NC + lax.axis_index("c")
    base = wid * _BPW

    pltpu.sync_copy(x_ref.at[:, pl.ds(base, _BPW)], idx_v)
    pltpu.sync_copy(w0_ref, w0_v)

    for f in range(_F):
        for gg in range(_G):
            sl0 = pl.ds(gg * _K, _K)
            idxq_v[f, sl0] = lax.shift_right_logical(idx_v[f, sl0], 4)

    emb_copies = [
        pltpu.async_copy(emb_ref.at[idx_v.at[f]], rows_v.at[f], sem)
        for f in range(_F)
    ]
    w1_copies = [
        pltpu.async_copy(w1_ref.at[idxq_v.at[f]], w1r_v.at[pl.ds(f * _BPW, _BPW)], wsem)
        for f in range(_F)
    ]
    for c in emb_copies:
        c.wait()
    for c in w1_copies:
        c.wait()

    w0s = w0_v[...]
    lane = lax.iota(jnp.int32, _K)

    def gbody(g, carry):
        for jj in range(_K):
            j = g * _K + jj
            s = rows_v[0, j]
            q = s * s
            for f in range(1, _F):
                v = rows_v[f, j]
                s = s + v
                q = q + v * v
            r = s * s - q
            plsc.store_scatter(rbuf_v, [lane * _K + jj], r)
        sl = pl.ds(g * _K, _K)
        fm2 = rbuf_v[pl.ds(0, _K)]
        for k in range(1, _K):
            fm2 = fm2 + rbuf_v[pl.ds(k * _K, _K)]
        col16 = g * _K + lane
        w1s = jnp.zeros((_K,), jnp.float32)
        for f in range(_F):
            rem16 = jnp.bitwise_and(idx_v[f, sl], 15)
            w1s = w1s + plsc.load_gather(w1r_v, [f * _BPW + col16, rem16])
        fm = w0s + w1s + 0.5 * fm2
        out_v[0, sl] = fm
        out_v[1, sl] = 1.0 / (1.0 + jnp.exp(-fm))
        return carry

    lax.fori_loop(0, _G, gbody, 0)

    pltpu.sync_copy(out_v.at[0], fm_ref.at[pl.ds(base, _BPW)])
    pltpu.sync_copy(out_v.at[1], prob_ref.at[pl.ds(base, _BPW)])


def kernel(x, emb_v, w1, w0):
    x32 = x.astype(jnp.int32)
    w1q = w1.reshape(-1, _K)
    w0v = jnp.broadcast_to(w0.astype(jnp.float32).reshape(1), (_K,))
    mesh = plsc.VectorSubcoreMesh(core_axis_name="c", subcore_axis_name="s")

    embP = pl.kernel(
        _transpose_body,
        out_type=jax.ShapeDtypeStruct((_H * _K,), jnp.float32),
        mesh=mesh,
        compiler_params=pltpu.CompilerParams(
            needs_layout_passes=False, use_tc_tiling_on_sc=True),
        scratch_types=[
            pltpu.VMEM((8, _RUNROWS), jnp.float32),
            pltpu.VMEM((8, _RUNROWS), jnp.float32),
            pltpu.VMEM((8, _RUNROWS), jnp.float32),
            pltpu.VMEM((8, _RUNROWS), jnp.float32),
            pltpu.VMEM((_RUNROWS * _K,), jnp.float32),
            pltpu.VMEM((_RUNROWS * _K,), jnp.float32),
            pltpu.SemaphoreType.DMA,
            pltpu.SemaphoreType.DMA,
            pltpu.SemaphoreType.DMA,
            pltpu.SemaphoreType.DMA,
        ],
    )(emb_v.T, emb_v[_H - _LAST:, :].reshape(_LAST * _K))
    emb_rm = embP.reshape(_H, _K)

    fm_flat, prob_flat = pl.kernel(
        _fm_body,
        out_type=(
            jax.ShapeDtypeStruct((_B,), jnp.float32),
            jax.ShapeDtypeStruct((_B,), jnp.float32),
        ),
        mesh=mesh,
        compiler_params=pltpu.CompilerParams(
            needs_layout_passes=False, use_tc_tiling_on_sc=False),
        scratch_types=[
            pltpu.VMEM((_F, _BPW), jnp.int32),
            pltpu.VMEM((_F, _BPW), jnp.int32),
            pltpu.VMEM((_F, _BPW, _K), jnp.float32),
            pltpu.VMEM((_F * _BPW, _K), jnp.float32),
            pltpu.VMEM((_K,), jnp.float32),
            pltpu.VMEM((_K * _K,), jnp.float32),
            pltpu.VMEM((2, _BPW), jnp.float32),
            pltpu.SemaphoreType.DMA,
            pltpu.SemaphoreType.DMA,
        ],
    )(x32, emb_rm, w1q, w0v)
    return fm_flat.reshape(_B, 1), prob_flat.reshape(_B, 1)


# speedup vs baseline: 1.0223x; 1.0223x over previous
import jax
import jax.numpy as jnp
from jax import lax
from jax.experimental import pallas as pl
from jax.experimental.pallas import tpu as pltpu
from jax.experimental.pallas import tpu_sc as plsc

_F = 26
_K = 16
_B = 4096
_H = 1000000
_NC = 2
_NS = 16
_NW = _NC * _NS
_BPW = _B // _NW
_G = _BPW // _K

_RUNROWS = 2048
_NFULL = _H // _RUNROWS
_TAILROWS = 512
_LAST = 64
_TAILWID = _NFULL % _NW


def _transpose_body(src_ref, tail_ref, out_ref, in0a, in1a,
                    ob0, ob1, is0, is1, os0, os1):
    wid = lax.axis_index("s") * _NC + lax.axis_index("c")
    lane = lax.iota(jnp.int32, _K)
    lanex16 = lane * _K

    def transpose_panel(nrows, inb, ob):
        def blk(c0i, carry):
            base = c0i * 256
            for k in range(_K):
                v = inb[k, pl.ds(c0i * _K, _K)]
                plsc.store_scatter(ob, [lanex16 + (base + k)], v)
            return carry
        lax.fori_loop(0, nrows // _K, blk, 0, unroll=2)

    inbufs = (in0a, in1a)
    obufs = (ob0, ob1)
    isems = (is0, is1)
    osems = (os0, os1)
    nj = (_NFULL + _NW - 1) // _NW

    def start_in(run, b):
        @pl.when(run < _NFULL)
        def _():
            pltpu.async_copy(
                src_ref.at[:, pl.ds(run * _RUNROWS, _RUNROWS)],
                inbufs[b], isems[b])

    def do_phase(j, b):
        run = wid + _NW * j
        nxt = run + _NW

        @pl.when(nxt < _NFULL)
        def _():
            pltpu.async_copy(
                src_ref.at[:, pl.ds(nxt * _RUNROWS, _RUNROWS)],
                inbufs[1 - b], isems[1 - b])

        if j >= 2:
            @pl.when(run - 2 * _NW < _NFULL)
            def _():
                pltpu.make_async_copy(
                    obufs[b],
                    out_ref.at[pl.ds((run - 2 * _NW) * _RUNROWS * _K,
                                     _RUNROWS * _K)], osems[b]).wait()

        @pl.when(run < _NFULL)
        def _():
            pltpu.make_async_copy(
                src_ref.at[:, pl.ds(run * _RUNROWS, _RUNROWS)],
                inbufs[b], isems[b]).wait()
            transpose_panel(_RUNROWS, inbufs[b], obufs[b])
            pltpu.async_copy(
                obufs[b], out_ref.at[pl.ds(run * _RUNROWS * _K, _RUNROWS * _K)],
                osems[b])

    start_in(wid, 0)
    for j in range(nj):
        do_phase(j, j % 2)
    for j in (nj - 2, nj - 1):
        run = wid + _NW * j

        @pl.when(run < _NFULL)
        def _():
            pltpu.make_async_copy(
                obufs[j % 2],
                out_ref.at[pl.ds(run * _RUNROWS * _K, _RUNROWS * _K)],
                osems[j % 2]).wait()

    @pl.when(wid == _TAILWID)
    def _tail():
        r0 = _NFULL * _RUNROWS
        pltpu.async_copy(
            src_ref.at[:, pl.ds(r0, _TAILROWS)],
            in0a.at[:, pl.ds(0, _TAILROWS)], is0).wait()
        transpose_panel(_TAILROWS, in0a, ob0)
        pltpu.async_copy(
            ob0.at[pl.ds(0, _TAILROWS * _K)],
            out_ref.at[pl.ds(r0 * _K, _TAILROWS * _K)], os0).wait()
        pltpu.async_copy(tail_ref, ob1.at[pl.ds(0, _LAST * _K)], is1).wait()
        pltpu.async_copy(
            ob1.at[pl.ds(0, _LAST * _K)],
            out_ref.at[pl.ds((_H - _LAST) * _K, _LAST * _K)], os1).wait()


def _fm_body(x_ref, emb_ref, w1_ref, w0_ref, fm_ref, prob_ref,
             idx_v, idxq_v, rows_v, w1r_v, w0_v, rbuf_v, out_v, sem, wsem):
    wid = lax.axis_index("s") * _NC + lax.axis_index("c")
    base = wid * _BPW

    pltpu.sync_copy(x_ref.at[:, pl.ds(base, _BPW)], idx_v)
    pltpu.sync_copy(w0_ref, w0_v)

    for f in range(_F):
        for gg in range(_G):
            sl0 = pl.ds(gg * _K, _K)
            idxq_v[f, sl0] = lax.shift_right_logical(idx_v[f, sl0], 4)

    emb_copies = [
        pltpu.async_copy(emb_ref.at[idx_v.at[f]], rows_v.at[f], sem)
        for f in range(_F)
    ]
    w1_copies = [
        pltpu.async_copy(w1_ref.at[idxq_v.at[f]], w1r_v.at[pl.ds(f * _BPW, _BPW)], wsem)
        for f in range(_F)
    ]
    for c in emb_copies:
        c.wait()
    for c in w1_copies:
        c.wait()

    w0s = w0_v[...]
    lane = lax.iota(jnp.int32, _K)

    def gbody(g, carry):
        for jj in range(_K):
            j = g * _K + jj
            s = rows_v[0, j]
            q = s * s
            for f in range(1, _F):
                v = rows_v[f, j]
                s = s + v
                q = q + v * v
            r = s * s - q
            plsc.store_scatter(rbuf_v, [lane * _K + jj], r)
        sl = pl.ds(g * _K, _K)
        fm2 = rbuf_v[pl.ds(0, _K)]
        for k in range(1, _K):
            fm2 = fm2 + rbuf_v[pl.ds(k * _K, _K)]
        col16 = g * _K + lane
        w1s = jnp.zeros((_K,), jnp.float32)
        for f in range(_F):
            rem16 = jnp.bitwise_and(idx_v[f, sl], 15)
            w1s = w1s + plsc.load_gather(w1r_v, [f * _BPW + col16, rem16])
        fm = w0s + w1s + 0.5 * fm2
        out_v[0, sl] = fm
        out_v[1, sl] = 1.0 / (1.0 + jnp.exp(-fm))
        return carry

    lax.fori_loop(0, _G, gbody, 0)

    pltpu.sync_copy(out_v.at[0], fm_ref.at[pl.ds(base, _BPW)])
    pltpu.sync_copy(out_v.at[1], prob_ref.at[pl.ds(base, _BPW)])


def kernel(x, emb_v, w1, w0):
    x32 = x.astype(jnp.int32)
    w1q = w1.reshape(-1, _K)
    w0v = jnp.broadcast_to(w0.astype(jnp.float32).reshape(1), (_K,))
    mesh = plsc.VectorSubcoreMesh(core_axis_name="c", subcore_axis_name="s")

    embP = pl.kernel(
        _transpose_body,
        out_type=jax.ShapeDtypeStruct((_H * _K,), jnp.float32),
        mesh=mesh,
        compiler_params=pltpu.CompilerParams(
            needs_layout_passes=False, use_tc_tiling_on_sc=True),
        scratch_types=[
            pltpu.VMEM((_K, _RUNROWS), jnp.float32),
            pltpu.VMEM((_K, _RUNROWS), jnp.float32),
            pltpu.VMEM((_RUNROWS * _K,), jnp.float32),
            pltpu.VMEM((_RUNROWS * _K,), jnp.float32),
            pltpu.SemaphoreType.DMA,
            pltpu.SemaphoreType.DMA,
            pltpu.SemaphoreType.DMA,
            pltpu.SemaphoreType.DMA,
        ],
    )(emb_v.T, emb_v[_H - _LAST:, :].reshape(_LAST * _K))
    emb_rm = embP.reshape(_H, _K)

    fm_flat, prob_flat = pl.kernel(
        _fm_body,
        out_type=(
            jax.ShapeDtypeStruct((_B,), jnp.float32),
            jax.ShapeDtypeStruct((_B,), jnp.float32),
        ),
        mesh=mesh,
        compiler_params=pltpu.CompilerParams(
            needs_layout_passes=False, use_tc_tiling_on_sc=False),
        scratch_types=[
            pltpu.VMEM((_F, _BPW), jnp.int32),
            pltpu.VMEM((_F, _BPW), jnp.int32),
            pltpu.VMEM((_F, _BPW, _K), jnp.float32),
            pltpu.VMEM((_F * _BPW, _K), jnp.float32),
            pltpu.VMEM((_K,), jnp.float32),
            pltpu.VMEM((_K * _K,), jnp.float32),
            pltpu.VMEM((2, _BPW), jnp.float32),
            pltpu.SemaphoreType.DMA,
            pltpu.SemaphoreType.DMA,
        ],
    )(x32, emb_rm, w1q, w0v)
    return fm_flat.reshape(_B, 1), prob_flat.reshape(_B, 1)


# speedup vs baseline: 1.4865x; 1.4540x over previous
import jax
import jax.numpy as jnp
from jax import lax
from jax.experimental import pallas as pl
from jax.experimental.pallas import tpu as pltpu
from jax.experimental.pallas import tpu_sc as plsc

_F = 26
_K = 16
_B = 4096
_H = 1000000
_NC = 2
_NS = 16
_NW = _NC * _NS
_BPW = _B // _NW
_G = _BPW // _K

_RUNROWS = 2048
_NFULL = _H // _RUNROWS
_TAILROWS = 512
_LAST = 64
_TAILWID = _NFULL % _NW


def _transpose_body(src_ref, tail_ref, out_ref, in0a, in1a,
                    ob0, ob1, is0, is1, os0, os1):
    wid = lax.axis_index("s") * _NC + lax.axis_index("c")
    lane = lax.iota(jnp.int32, _K)
    lanex16 = lane * _K

    def transpose_panel(nrows, inb, ob):
        @plsc.parallel_loop(0, nrows // _K, 1, unroll=4)
        def blk(c0i):
            base = c0i * 256
            for k in range(_K):
                v = inb[k, pl.ds(c0i * _K, _K)]
                plsc.store_scatter(ob, [lanex16 + (base + k)], v)

    inbufs = (in0a, in1a)
    obufs = (ob0, ob1)
    isems = (is0, is1)
    osems = (os0, os1)
    nj = (_NFULL + _NW - 1) // _NW

    def start_in(run, b):
        @pl.when(run < _NFULL)
        def _():
            pltpu.async_copy(
                src_ref.at[:, pl.ds(run * _RUNROWS, _RUNROWS)],
                inbufs[b], isems[b])

    def do_phase(j, b):
        run = wid + _NW * j
        nxt = run + _NW

        @pl.when(nxt < _NFULL)
        def _():
            pltpu.async_copy(
                src_ref.at[:, pl.ds(nxt * _RUNROWS, _RUNROWS)],
                inbufs[1 - b], isems[1 - b])

        if j >= 2:
            @pl.when(run - 2 * _NW < _NFULL)
            def _():
                pltpu.make_async_copy(
                    obufs[b],
                    out_ref.at[pl.ds((run - 2 * _NW) * _RUNROWS * _K,
                                     _RUNROWS * _K)], osems[b]).wait()

        @pl.when(run < _NFULL)
        def _():
            pltpu.make_async_copy(
                src_ref.at[:, pl.ds(run * _RUNROWS, _RUNROWS)],
                inbufs[b], isems[b]).wait()
            transpose_panel(_RUNROWS, inbufs[b], obufs[b])
            pltpu.async_copy(
                obufs[b], out_ref.at[pl.ds(run * _RUNROWS * _K, _RUNROWS * _K)],
                osems[b])

    start_in(wid, 0)
    for j in range(nj):
        do_phase(j, j % 2)
    for j in (nj - 2, nj - 1):
        run = wid + _NW * j

        @pl.when(run < _NFULL)
        def _():
            pltpu.make_async_copy(
                obufs[j % 2],
                out_ref.at[pl.ds(run * _RUNROWS * _K, _RUNROWS * _K)],
                osems[j % 2]).wait()

    @pl.when(wid == _TAILWID)
    def _tail():
        r0 = _NFULL * _RUNROWS
        pltpu.async_copy(
            src_ref.at[:, pl.ds(r0, _TAILROWS)],
            in0a.at[:, pl.ds(0, _TAILROWS)], is0).wait()
        transpose_panel(_TAILROWS, in0a, ob0)
        pltpu.async_copy(
            ob0.at[pl.ds(0, _TAILROWS * _K)],
            out_ref.at[pl.ds(r0 * _K, _TAILROWS * _K)], os0).wait()
        pltpu.async_copy(tail_ref, ob1.at[pl.ds(0, _LAST * _K)], is1).wait()
        pltpu.async_copy(
            ob1.at[pl.ds(0, _LAST * _K)],
            out_ref.at[pl.ds((_H - _LAST) * _K, _LAST * _K)], os1).wait()


def _fm_body(x_ref, emb_ref, w1_ref, w0_ref, fm_ref, prob_ref,
             idx_v, idxq_v, rows_v, w1r_v, w0_v, rbuf_v, out_v, sem, wsem):
    wid = lax.axis_index("s") * _NC + lax.axis_index("c")
    base = wid * _BPW

    pltpu.sync_copy(x_ref.at[:, pl.ds(base, _BPW)], idx_v)
    pltpu.sync_copy(w0_ref, w0_v)

    for f in range(_F):
        for gg in range(_G):
            sl0 = pl.ds(gg * _K, _K)
            idxq_v[f, sl0] = lax.shift_right_logical(idx_v[f, sl0], 4)

    emb_copies = [
        pltpu.async_copy(emb_ref.at[idx_v.at[f]], rows_v.at[f], sem)
        for f in range(_F)
    ]
    w1_copies = [
        pltpu.async_copy(w1_ref.at[idxq_v.at[f]], w1r_v.at[pl.ds(f * _BPW, _BPW)], wsem)
        for f in range(_F)
    ]
    for c in emb_copies:
        c.wait()
    for c in w1_copies:
        c.wait()

    w0s = w0_v[...]
    lane = lax.iota(jnp.int32, _K)

    def gbody(g, carry):
        for jj in range(_K):
            j = g * _K + jj
            s = rows_v[0, j]
            q = s * s
            for f in range(1, _F):
                v = rows_v[f, j]
                s = s + v
                q = q + v * v
            r = s * s - q
            plsc.store_scatter(rbuf_v, [lane * _K + jj], r)
        sl = pl.ds(g * _K, _K)
        fm2 = rbuf_v[pl.ds(0, _K)]
        for k in range(1, _K):
            fm2 = fm2 + rbuf_v[pl.ds(k * _K, _K)]
        col16 = g * _K + lane
        w1s = jnp.zeros((_K,), jnp.float32)
        for f in range(_F):
            rem16 = jnp.bitwise_and(idx_v[f, sl], 15)
            w1s = w1s + plsc.load_gather(w1r_v, [f * _BPW + col16, rem16])
        fm = w0s + w1s + 0.5 * fm2
        out_v[0, sl] = fm
        out_v[1, sl] = 1.0 / (1.0 + jnp.exp(-fm))
        return carry

    lax.fori_loop(0, _G, gbody, 0)

    pltpu.sync_copy(out_v.at[0], fm_ref.at[pl.ds(base, _BPW)])
    pltpu.sync_copy(out_v.at[1], prob_ref.at[pl.ds(base, _BPW)])


def kernel(x, emb_v, w1, w0):
    x32 = x.astype(jnp.int32)
    w1q = w1.reshape(-1, _K)
    w0v = jnp.broadcast_to(w0.astype(jnp.float32).reshape(1), (_K,))
    mesh = plsc.VectorSubcoreMesh(core_axis_name="c", subcore_axis_name="s")

    embP = pl.kernel(
        _transpose_body,
        out_type=jax.ShapeDtypeStruct((_H * _K,), jnp.float32),
        mesh=mesh,
        compiler_params=pltpu.CompilerParams(
            needs_layout_passes=False, use_tc_tiling_on_sc=True),
        scratch_types=[
            pltpu.VMEM((_K, _RUNROWS), jnp.float32),
            pltpu.VMEM((_K, _RUNROWS), jnp.float32),
            pltpu.VMEM((_RUNROWS * _K,), jnp.float32),
            pltpu.VMEM((_RUNROWS * _K,), jnp.float32),
            pltpu.SemaphoreType.DMA,
            pltpu.SemaphoreType.DMA,
            pltpu.SemaphoreType.DMA,
            pltpu.SemaphoreType.DMA,
        ],
    )(emb_v.T, emb_v[_H - _LAST:, :].reshape(_LAST * _K))
    emb_rm = embP.reshape(_H, _K)

    fm_flat, prob_flat = pl.kernel(
        _fm_body,
        out_type=(
            jax.ShapeDtypeStruct((_B,), jnp.float32),
            jax.ShapeDtypeStruct((_B,), jnp.float32),
        ),
        mesh=mesh,
        compiler_params=pltpu.CompilerParams(
            needs_layout_passes=False, use_tc_tiling_on_sc=False),
        scratch_types=[
            pltpu.VMEM((_F, _BPW), jnp.int32),
            pltpu.VMEM((_F, _BPW), jnp.int32),
            pltpu.VMEM((_F, _BPW, _K), jnp.float32),
            pltpu.VMEM((_F * _BPW, _K), jnp.float32),
            pltpu.VMEM((_K,), jnp.float32),
            pltpu.VMEM((_K * _K,), jnp.float32),
            pltpu.VMEM((2, _BPW), jnp.float32),
            pltpu.SemaphoreType.DMA,
            pltpu.SemaphoreType.DMA,
        ],
    )(x32, emb_rm, w1q, w0v)
    return fm_flat.reshape(_B, 1), prob_flat.reshape(_B, 1)
